# baseline (speedup 1.0000x reference)
import functools
import math

import jax
import jax.numpy as jnp
from jax import lax
from jax.experimental import pallas as pl
from jax.experimental.pallas import tpu as pltpu
from jax.experimental.pallas import tpu_sc as plsc

D_MODEL = 64
SCALE = math.sqrt(D_MODEL)

NUM_WORKERS = 32
BLOCK = 128
LANES = 16


def _make_kernel(n_rows):
    rows_per_worker = n_rows // NUM_WORKERS
    n_blocks = rows_per_worker // BLOCK
    mesh = plsc.VectorSubcoreMesh(core_axis_name="c", subcore_axis_name="s")

    @functools.partial(
        pl.kernel,
        out_type=jax.ShapeDtypeStruct((n_rows, D_MODEL), jnp.float32),
        mesh=mesh,
        scratch_types=[
            pltpu.VMEM((n_blocks, BLOCK), jnp.int32),
            pltpu.VMEM((2, BLOCK, D_MODEL), jnp.float32),
            pltpu.SemaphoreType.DMA,
        ],
        compiler_params=pltpu.CompilerParams(use_tc_tiling_on_sc=False),
    )
    def gather_scale(x_hbm, table_hbm, out_hbm, idx_v, rows_v, sem):
        cid = lax.axis_index("c")
        sid = lax.axis_index("s")
        wid = sid * 2 + cid
        base = wid * rows_per_worker
        pltpu.sync_copy(x_hbm.at[wid], idx_v)

        def block_body(g, carry):
            pltpu.async_copy(table_hbm.at[idx_v.at[g]], rows_v.at[0], sem).wait()

            def row_body(r, c2):
                for j in range(D_MODEL // LANES):
                    sl = pl.ds(j * LANES, LANES)
                    rows_v[0, r, sl] = rows_v[0, r, sl] * SCALE
                return c2

            lax.fori_loop(0, BLOCK, row_body, 0)
            pltpu.sync_copy(
                rows_v.at[0], out_hbm.at[pl.ds(base + g * BLOCK, BLOCK)]
            )
            return carry

        lax.fori_loop(0, n_blocks, block_body, 0)

    return gather_scale


def kernel(x, table):
    b, s = x.shape
    n_rows = b * s
    x_blocked = x.reshape(NUM_WORKERS, n_rows // (NUM_WORKERS * BLOCK), BLOCK)
    out = _make_kernel(n_rows)(x_blocked, table)
    return out.reshape(b, s, D_MODEL)


# speedup vs baseline: 1.2060x; 1.2060x over previous
import functools
import math

import jax
import jax.numpy as jnp
from jax import lax
from jax.experimental import pallas as pl
from jax.experimental.pallas import tpu as pltpu
from jax.experimental.pallas import tpu_sc as plsc

D_MODEL = 64
SCALE = math.sqrt(D_MODEL)

NUM_WORKERS = 32
BLOCK = 128
NBUF = 4
LANES = 16


def _make_kernel(n_rows):
    rows_per_worker = n_rows // NUM_WORKERS
    n_blocks = rows_per_worker // BLOCK
    n_groups = n_blocks // NBUF
    assert n_groups % 2 == 0
    mesh = plsc.VectorSubcoreMesh(core_axis_name="c", subcore_axis_name="s")

    @functools.partial(
        pl.kernel,
        out_type=jax.ShapeDtypeStruct((n_rows, D_MODEL), jnp.float32),
        mesh=mesh,
        scratch_types=[
            pltpu.VMEM((n_blocks, BLOCK), jnp.int32),
            pltpu.VMEM((2, NBUF, BLOCK, D_MODEL), jnp.float32),
            pltpu.SemaphoreType.DMA,
            pltpu.SemaphoreType.DMA,
        ],
        compiler_params=pltpu.CompilerParams(use_tc_tiling_on_sc=False),
    )
    def gather_scale(x_hbm, table_hbm, out_hbm, idx_v, rows_v, gsem, ssem):
        cid = lax.axis_index("c")
        sid = lax.axis_index("s")
        wid = sid * 2 + cid
        base = wid * rows_per_worker
        pltpu.sync_copy(x_hbm.at[wid], idx_v)

        def gather(go, q):
            for b in range(NBUF):
                pltpu.async_copy(
                    table_hbm.at[idx_v.at[go * NBUF + b]], rows_v.at[q, b], gsem
                )

        def gather_wait(go, q):
            for b in range(NBUF):
                pltpu.make_async_copy(
                    table_hbm.at[idx_v.at[go * NBUF + b]], rows_v.at[q, b], gsem
                ).wait()

        def store(go, q):
            for b in range(NBUF):
                pltpu.async_copy(
                    rows_v.at[q, b],
                    out_hbm.at[pl.ds(base + (go * NBUF + b) * BLOCK, BLOCK)],
                    ssem,
                )

        def store_wait(go, q):
            for b in range(NBUF):
                pltpu.make_async_copy(
                    rows_v.at[q, b],
                    out_hbm.at[pl.ds(base + (go * NBUF + b) * BLOCK, BLOCK)],
                    ssem,
                ).wait()

        def scale(q):
            for b in range(NBUF):
                def row_body(r, c):
                    for rr in range(2):
                        for j in range(D_MODEL // LANES):
                            sl = pl.ds(j * LANES, LANES)
                            rows_v[q, b, 2 * r + rr, sl] = (
                                rows_v[q, b, 2 * r + rr, sl] * SCALE
                            )
                    return c
                lax.fori_loop(0, BLOCK // 2, row_body, 0)

        gather(0, 0)

        def super_body(gs, carry):
            for q in range(2):
                go = gs * 2 + q
                gather_wait(go, q)
                @pl.when(go >= 1)
                def _():
                    store_wait(go - 1, 1 - q)
                @pl.when(go < n_groups - 1)
                def _():
                    gather(go + 1, 1 - q)
                scale(q)
                store(go, q)
            return carry

        lax.fori_loop(0, n_groups // 2, super_body, 0)
        store_wait(n_groups - 1, 1)

    return gather_scale


def kernel(x, table):
    b, s = x.shape
    n_rows = b * s
    x_blocked = x.reshape(NUM_WORKERS, n_rows // (NUM_WORKERS * BLOCK), BLOCK)
    out = _make_kernel(n_rows)(x_blocked, table)
    return out.reshape(b, s, D_MODEL)
